# baseline (speedup 1.0000x reference)
import functools
import jax
import jax.numpy as jnp
from jax.experimental import pallas as pl

S = 64.0
MEAN = 0.5
SIGMA = 0.05


def _body(lab_ref, ce_ref, se_ref, x_ref, o_ref, *, bc, sub):
    j = pl.program_id(1)
    br = x_ref.shape[0]
    lab = lab_ref[0, 0, :][:, None]
    ceS = ce_ref[0, 0, :][:, None]
    seS = se_ref[0, 0, :][:, None]
    for s in range(bc // sub):
        x = x_ref[:, pl.ds(s * sub, sub)]
        cols = (jax.lax.broadcasted_iota(jnp.int32, (br, sub), 1)
                + (j * bc + s * sub))
        m = cols == lab
        fix = x * ceS - jnp.sqrt(jnp.maximum(1.0 - x * x, 0.0)) * seS
        o_ref[:, pl.ds(s * sub, sub)] = jnp.where(m, fix, x * S)


def kernel(logits, labels):
    B, C = logits.shape
    BR = min(1024, B)
    BC = 2048
    SUB = 512
    grid_r = pl.cdiv(B, BR)
    grid_c = pl.cdiv(C, BC)

    elastic = jax.random.normal(jax.random.key(42), (B,), dtype=logits.dtype)
    elastic = elastic * SIGMA + MEAN
    ce = (jnp.cos(elastic) * S).reshape(grid_r, 1, BR)
    se = (jnp.sin(elastic) * S).reshape(grid_r, 1, BR)
    labs = labels.astype(jnp.int32).reshape(grid_r, 1, BR)

    body = functools.partial(_body, bc=BC, sub=SUB)

    return pl.pallas_call(
        body,
        grid=(grid_r, grid_c),
        in_specs=[
            pl.BlockSpec((1, 1, BR), lambda i, j: (i, 0, 0)),
            pl.BlockSpec((1, 1, BR), lambda i, j: (i, 0, 0)),
            pl.BlockSpec((1, 1, BR), lambda i, j: (i, 0, 0)),
            pl.BlockSpec((BR, BC), lambda i, j: (i, j)),
        ],
        out_specs=pl.BlockSpec((BR, BC), lambda i, j: (i, j)),
        out_shape=jax.ShapeDtypeStruct((B, C), logits.dtype),
    )(labs, ce, se, logits)
